# baseline (speedup 1.0000x reference)
import functools

import jax
import jax.numpy as jnp
from jax import lax
from jax.experimental import pallas as pl
from jax.experimental.pallas import tpu as pltpu
from jax.experimental.pallas import tpu_sc as plsc

B = 4
C = 32
PLANE_DIMS = ((0, 2), (0, 1), (1, 2))
DENOM = 1.0 + 0.1 + 10e-6
CLIP = float(1.0 - 10e-6)


def _cdiv(a, b):
    return (a + b - 1) // b


def _make_scale_kernel(N, R):
    R2 = R * R
    NB = _cdiv(R2 + 128, 2048) * 2048
    NR = NB // 128
    NRT = NR // 16
    ZQ = NB // 2048
    NSUB = _cdiv(N, 128)
    TAIL = N - (NSUB - 1) * 128
    QMAX = _cdiv(NSUB, 16)
    NROWS = R2 // 128
    QROWS = _cdiv(NROWS, 16)
    NA = min(NR, 128)
    NBROWS = NR - NA

    mesh = plsc.VectorSubcoreMesh(core_axis_name="c", subcore_axis_name="s",
                                  num_cores=2, num_subcores=16)

    scratch = [
        pltpu.VMEM_SHARED((NB, C), jnp.bfloat16),
        pltpu.VMEM_SHARED((NB, C), jnp.bfloat16),
        pltpu.VMEM_SHARED((NB, C), jnp.bfloat16),
        pltpu.VMEM_SHARED((NR, 128), jnp.float32),
        pltpu.VMEM_SHARED((NR, 128), jnp.float32),
        pltpu.VMEM_SHARED((NR, 128), jnp.float32),
        pltpu.VMEM((NR, 128), jnp.float32),
        pltpu.VMEM((NR, 128), jnp.float32),
        pltpu.VMEM((NR, 128), jnp.float32),
        pltpu.VMEM((128, C), jnp.bfloat16),
        pltpu.VMEM((384,), jnp.float32),
        pltpu.VMEM((128, C), jnp.bfloat16),
        pltpu.VMEM((384,), jnp.float32),
        pltpu.VMEM((128,), jnp.int32),
        pltpu.VMEM((128,), jnp.int32),
        pltpu.VMEM((128,), jnp.int32),
        pltpu.VMEM((128,), jnp.int32),
        pltpu.VMEM((128,), jnp.int32),
        pltpu.VMEM((128,), jnp.int32),
        pltpu.VMEM((NA,), jnp.int32),
        pltpu.VMEM((max(NBROWS, 16),), jnp.int32),
        pltpu.VMEM((128, C), jnp.bfloat16),
        pltpu.VMEM((NRT, 128), jnp.float32),
        pltpu.VMEM((1, 128), jnp.float32),
        pltpu.VMEM((128, C), jnp.bfloat16),
        pltpu.VMEM((C, 128), jnp.float32),
        pltpu.VMEM((128,), jnp.float32),
        pltpu.SemaphoreType.DMA,
        pltpu.SemaphoreType.DMA,
        pltpu.SemaphoreType.DMA,
    ]

    out_type = tuple(
        jax.ShapeDtypeStruct((B, C, R2), jnp.float32) for _ in range(3)
    )

    @functools.partial(
        pl.kernel, out_type=out_type, mesh=mesh,
        compiler_params=pltpu.CompilerParams(needs_layout_passes=False,
                                             use_tc_tiling_on_sc=False),
        scratch_types=scratch)
    def scale_kernel(pf, cr, oxz, oxy, oyz,
                     acc0, acc1, acc2, cnt0, cnt1, cnt2,
                     hist0, hist1, hist2, pfbuf, crbuf, pfbuf2, crbuf2,
                     i0, i1, i2, i3, i4, i5, ia, ib, zbuf, zcbuf,
                     cntrow, abuf, obuf, rbuf, lsemA, lsemB, ssem):
        cid = lax.axis_index("c")
        sid = lax.axis_index("s")
        i16 = lax.iota(jnp.int32, 16)
        evens = i16 * 2
        odds = i16 * 2 + 1
        one16 = jnp.full((16,), 1.0, jnp.float32)
        zero16 = jnp.zeros((16,), jnp.float32)
        zero32b = jnp.zeros((32,), jnp.bfloat16)
        accs = (acc0, acc1, acc2)
        cnts = (cnt0, cnt1, cnt2)
        hists = (hist0, hist1, hist2)
        ibufs = (i0, i1, i2)
        ibufs2 = (i3, i4, i5)
        outs = (oxz, oxy, oyz)

        @pl.loop(0, 128)
        def _(r):
            zbuf[r, pl.ds(0, 32)] = zero32b

        @pl.loop(0, NRT)
        def _(r):
            for g in range(8):
                zcbuf[r, pl.ds(g * 16, 16)] = zero16

        for g in range(NA // 16):
            ia[pl.ds(g * 16, 16)] = i16 + g * 16
        for g in range(max(NBROWS, 16) // 16):
            ib[pl.ds(g * 16, 16)] = i16 + (NA + g * 16)

        def compute_group(crb, j, n_valid):
            pos3 = (i16 + j * 16) * 3
            xs = []
            for d in range(3):
                v = plsc.load_gather(crb, [pos3 + d])
                u = v + 0.5
                u = jnp.minimum(jnp.maximum(u, 0.0), CLIP)
                xs.append((u * float(R)).astype(jnp.int32))
            idxs = [xs[a] + R * xs[b] for (a, b) in PLANE_DIMS]
            if n_valid < 128:
                pos = i16 + j * 16
                idxs = [jnp.where(pos < n_valid, ix, R2 + pos)
                        for ix in idxs]
            return idxs

        @pl.loop(0, 2)
        def _(kb):
            b = 2 * cid + kb

            for q in range(ZQ):
                base = (sid * ZQ + q) * 128
                for a in accs:
                    pltpu.sync_copy(zbuf, a.at[pl.ds(base, 128)])
            for cn in cnts:
                pltpu.sync_copy(zcbuf, cn.at[pl.ds(sid * NRT, NRT)])

            @pl.loop(0, NR)
            def _(r):
                for g in range(8):
                    sl = pl.ds(g * 16, 16)
                    hist0[r, sl] = zero16
                    hist1[r, sl] = zero16
                    hist2[r, sl] = zero16

            plsc.subcore_barrier()

            def issue_load(k, pfb, crb, lsem):
                o = k * 128

                @pl.when(k < NSUB - 1)
                def _():
                    pltpu.async_copy(pf.at[b, pl.ds(o, 128), :], pfb, lsem)
                    pltpu.async_copy(cr.at[b, pl.ds(3 * o, 384)], crb, lsem)

                @pl.when(k == NSUB - 1)
                def _():
                    pltpu.async_copy(pf.at[b, pl.ds(o, TAIL), :],
                                     pfb.at[pl.ds(0, TAIL)], lsem)
                    pltpu.async_copy(cr.at[b, pl.ds(3 * o, 3 * TAIL)],
                                     crb.at[pl.ds(0, 3 * TAIL)], lsem)

            def wait_load(k, pfb, crb, lsem):
                o = k * 128

                @pl.when(k < NSUB - 1)
                def _():
                    pltpu.make_async_copy(
                        pf.at[b, pl.ds(o, 128), :], pfb, lsem).wait()
                    pltpu.make_async_copy(
                        cr.at[b, pl.ds(3 * o, 384)], crb, lsem).wait()

                @pl.when(k == NSUB - 1)
                def _():
                    pltpu.make_async_copy(
                        pf.at[b, pl.ds(o, TAIL), :],
                        pfb.at[pl.ds(0, TAIL)], lsem).wait()
                    pltpu.make_async_copy(
                        cr.at[b, pl.ds(3 * o, 3 * TAIL)],
                        crb.at[pl.ds(0, 3 * TAIL)], lsem).wait()

            def sub_iter(k, pfb, crb, ibs, lsem_here, pfo, cro, lsem_next):
                @pl.when(k < NSUB)
                def _():
                    issue_load(k + 16, pfo, cro, lsem_next)
                    wait_load(k, pfb, crb, lsem_here)

                    def compute_chunk(n_valid):
                        @pl.loop(0, 8)
                        def _(j):
                            idxs = compute_group(crb, j, n_valid)
                            for ibf, h, iv in zip(ibs, hists, idxs):
                                ibf[pl.ds(j * 16, 16)] = iv
                                row = lax.shift_right_logical(iv, 7)
                                col = lax.bitwise_and(iv, 127)
                                plsc.addupdate_scatter(h, [row, col], one16)

                    @pl.when(k < NSUB - 1)
                    def _():
                        compute_chunk(128)

                    @pl.when(k == NSUB - 1)
                    def _():
                        compute_chunk(TAIL)

                    descs = [pltpu.async_copy(pfb, a.at[ib_], ssem, add=True)
                             for a, ib_ in zip(accs, ibs)]
                    for dd in descs:
                        dd.wait()

            issue_load(sid, pfbuf, crbuf, lsemA)

            @pl.loop(0, (QMAX + 1) // 2)
            def _(q2):
                k = sid + q2 * 32
                sub_iter(k, pfbuf, crbuf, ibufs, lsemA,
                         pfbuf2, crbuf2, lsemB)
                sub_iter(k + 16, pfbuf2, crbuf2, ibufs2, lsemB,
                         pfbuf, crbuf, lsemA)

            plsc.subcore_barrier()

            for h, cn in zip(hists, cnts):
                pltpu.sync_copy(h.at[pl.ds(0, NA)], cn.at[ia], add=True)
                if NBROWS > 0:
                    pltpu.sync_copy(h.at[pl.ds(NA, NBROWS)], cn.at[ib],
                                    add=True)

            plsc.subcore_barrier()

            @pl.loop(0, QROWS)
            def _(qr):
                r = sid + qr * 16

                @pl.when(r < NROWS)
                def _():
                    for a, cn, ou in zip(accs, cnts, outs):
                        pltpu.sync_copy(a.at[pl.ds(r * 128, 128)], abuf)
                        pltpu.sync_copy(cn.at[pl.ds(r, 1)], cntrow)

                        @pl.loop(0, 8)
                        def _(g):
                            cv = cntrow[0, pl.ds(g * 16, 16)]
                            rbuf[pl.ds(g * 16, 16)] = one16 / jnp.maximum(
                                cv, one16)

                        @pl.loop(0, 128)
                        def _(bn):
                            row = abuf[bn, pl.ds(0, C)]
                            e, o = plsc.unpack(
                                row, format=plsc.PackFormat.INTERLEAVED)
                            rec = plsc.load_gather(
                                rbuf, [jnp.broadcast_to(bn, (16,))])
                            col = jnp.broadcast_to(bn, (16,))
                            plsc.store_scatter(obuf, [evens, col], e * rec)
                            plsc.store_scatter(obuf, [odds, col], o * rec)

                        pltpu.sync_copy(obuf,
                                        ou.at[b, :, pl.ds(r * 128, 128)])

            plsc.subcore_barrier()

    return scale_kernel


_SCALES = ((100000, 128), (50000, 96), (25000, 64))


@functools.lru_cache(maxsize=None)
def _scale_kernels():
    return tuple(_make_scale_kernel(N, R) for (N, R) in _SCALES)


def kernel(p_f0, coord0, p_f1, coord1, p_f2, coord2):
    pfs = (p_f0, p_f1, p_f2)
    cds = (coord0, coord1, coord2)
    kernels = _scale_kernels()
    outs = []
    for i, (N, R) in enumerate(_SCALES):
        cr = (cds[i] / jnp.float32(DENOM)).reshape(B, 3 * N)
        oxz, oxy, oyz = kernels[i](pfs[i].astype(jnp.bfloat16), cr)
        for o in (oxz, oxy, oyz):
            outs.append(o.reshape(B, C, R, R))
    return tuple(outs)


# speedup vs baseline: 1.2906x; 1.2906x over previous
import functools

import jax
import jax.numpy as jnp
from jax import lax
from jax.experimental import pallas as pl
from jax.experimental.pallas import tpu as pltpu
from jax.experimental.pallas import tpu_sc as plsc

B = 4
C = 32
PLANE_DIMS = ((0, 2), (0, 1), (1, 2))
DENOM = 1.0 + 0.1 + 10e-6
CLIP = float(1.0 - 10e-6)


def _cdiv(a, b):
    return (a + b - 1) // b


def _make_scale_kernel(N, R):
    R2 = R * R
    NB = _cdiv(R2 + 128, 2048) * 2048
    NR = NB // 128
    NRT = NR // 16
    ZQ = NB // 2048
    NSUB = _cdiv(N, 128)
    TAIL = N - (NSUB - 1) * 128
    QMAX = _cdiv(NSUB, 16)
    NROWS = R2 // 128
    QROWS = _cdiv(NROWS, 16)
    NA = min(NR, 128)
    NBROWS = NR - NA

    mesh = plsc.VectorSubcoreMesh(core_axis_name="c", subcore_axis_name="s",
                                  num_cores=2, num_subcores=16)

    scratch = [
        pltpu.VMEM_SHARED((NB, C), jnp.bfloat16),
        pltpu.VMEM_SHARED((NB, C), jnp.bfloat16),
        pltpu.VMEM_SHARED((NB, C), jnp.bfloat16),
        pltpu.VMEM_SHARED((NR, 128), jnp.float32),
        pltpu.VMEM_SHARED((NR, 128), jnp.float32),
        pltpu.VMEM_SHARED((NR, 128), jnp.float32),
        pltpu.VMEM((NR, 128), jnp.float32),
        pltpu.VMEM((NR, 128), jnp.float32),
        pltpu.VMEM((NR, 128), jnp.float32),
        pltpu.VMEM((128, C), jnp.bfloat16),
        pltpu.VMEM((3, 128), jnp.float32),
        pltpu.VMEM((128, C), jnp.bfloat16),
        pltpu.VMEM((3, 128), jnp.float32),
        pltpu.VMEM((128,), jnp.int32),
        pltpu.VMEM((128,), jnp.int32),
        pltpu.VMEM((128,), jnp.int32),
        pltpu.VMEM((128,), jnp.int32),
        pltpu.VMEM((128,), jnp.int32),
        pltpu.VMEM((128,), jnp.int32),
        pltpu.VMEM((NA,), jnp.int32),
        pltpu.VMEM((max(NBROWS, 16),), jnp.int32),
        pltpu.VMEM((128, C), jnp.bfloat16),
        pltpu.VMEM((NRT, 128), jnp.float32),
        pltpu.VMEM((1, 128), jnp.float32),
        pltpu.VMEM((128, C), jnp.bfloat16),
        pltpu.VMEM((C, 128), jnp.float32),
        pltpu.VMEM((128,), jnp.float32),
        pltpu.SemaphoreType.DMA,
        pltpu.SemaphoreType.DMA,
        pltpu.SemaphoreType.DMA,
    ]

    out_type = tuple(
        jax.ShapeDtypeStruct((B, C, R2), jnp.float32) for _ in range(3)
    )

    @functools.partial(
        pl.kernel, out_type=out_type, mesh=mesh,
        compiler_params=pltpu.CompilerParams(needs_layout_passes=False,
                                             use_tc_tiling_on_sc=False),
        scratch_types=scratch)
    def scale_kernel(pf, cr, oxz, oxy, oyz,
                     acc0, acc1, acc2, cnt0, cnt1, cnt2,
                     hist0, hist1, hist2, pfbuf, crbuf, pfbuf2, crbuf2,
                     i0, i1, i2, i3, i4, i5, ia, ib, zbuf, zcbuf,
                     cntrow, abuf, obuf, rbuf, lsemA, lsemB, ssem):
        cid = lax.axis_index("c")
        sid = lax.axis_index("s")
        i16 = lax.iota(jnp.int32, 16)
        evens = i16 * 2
        odds = i16 * 2 + 1
        one16 = jnp.full((16,), 1.0, jnp.float32)
        zero16 = jnp.zeros((16,), jnp.float32)
        zero32b = jnp.zeros((32,), jnp.bfloat16)
        accs = (acc0, acc1, acc2)
        cnts = (cnt0, cnt1, cnt2)
        hists = (hist0, hist1, hist2)
        ibufs = (i0, i1, i2)
        ibufs2 = (i3, i4, i5)
        outs = (oxz, oxy, oyz)

        @pl.loop(0, 128)
        def _(r):
            zbuf[r, pl.ds(0, 32)] = zero32b

        @pl.loop(0, NRT)
        def _(r):
            for g in range(8):
                zcbuf[r, pl.ds(g * 16, 16)] = zero16

        for g in range(NA // 16):
            ia[pl.ds(g * 16, 16)] = i16 + g * 16
        for g in range(max(NBROWS, 16) // 16):
            ib[pl.ds(g * 16, 16)] = i16 + (NA + g * 16)

        def compute_group(crb, j, n_valid):
            xs = []
            for d in range(3):
                v = crb[d, pl.ds(j * 16, 16)]
                u = v + 0.5
                u = jnp.minimum(jnp.maximum(u, 0.0), CLIP)
                xs.append((u * float(R)).astype(jnp.int32))
            idxs = [xs[a] + R * xs[b] for (a, b) in PLANE_DIMS]
            if n_valid < 128:
                pos = i16 + j * 16
                idxs = [jnp.where(pos < n_valid, ix, R2 + pos)
                        for ix in idxs]
            return idxs

        @pl.loop(0, 2)
        def _(kb):
            b = 2 * cid + kb

            zds = []
            for q in range(ZQ):
                base = (sid * ZQ + q) * 128
                for a in accs:
                    zds.append(pltpu.async_copy(
                        zbuf, a.at[pl.ds(base, 128)], lsemA))
            for cn in cnts:
                zds.append(pltpu.async_copy(
                    zcbuf, cn.at[pl.ds(sid * NRT, NRT)], lsemA))
            for dd in zds:
                dd.wait()

            @pl.loop(0, NR)
            def _(r):
                for g in range(8):
                    sl = pl.ds(g * 16, 16)
                    hist0[r, sl] = zero16
                    hist1[r, sl] = zero16
                    hist2[r, sl] = zero16

            plsc.subcore_barrier()

            def issue_load(k, pfb, crb, lsem):
                o = k * 128

                @pl.when(k < NSUB - 1)
                def _():
                    pltpu.async_copy(pf.at[b, pl.ds(o, 128), :], pfb, lsem)
                    pltpu.async_copy(cr.at[b, :, pl.ds(o, 128)], crb, lsem)

                @pl.when(k == NSUB - 1)
                def _():
                    pltpu.async_copy(pf.at[b, pl.ds(o, TAIL), :],
                                     pfb.at[pl.ds(0, TAIL)], lsem)
                    pltpu.async_copy(cr.at[b, :, pl.ds(o, TAIL)],
                                     crb.at[:, pl.ds(0, TAIL)], lsem)

            def wait_load(k, pfb, crb, lsem):
                o = k * 128

                @pl.when(k < NSUB - 1)
                def _():
                    pltpu.make_async_copy(
                        pf.at[b, pl.ds(o, 128), :], pfb, lsem).wait()
                    pltpu.make_async_copy(
                        cr.at[b, :, pl.ds(o, 128)], crb, lsem).wait()

                @pl.when(k == NSUB - 1)
                def _():
                    pltpu.make_async_copy(
                        pf.at[b, pl.ds(o, TAIL), :],
                        pfb.at[pl.ds(0, TAIL)], lsem).wait()
                    pltpu.make_async_copy(
                        cr.at[b, :, pl.ds(o, TAIL)],
                        crb.at[:, pl.ds(0, TAIL)], lsem).wait()

            def sub_iter(k, pfb, crb, ibs, lsem_here, pfo, cro, lsem_next):
                @pl.when(k < NSUB)
                def _():
                    issue_load(k + 16, pfo, cro, lsem_next)
                    wait_load(k, pfb, crb, lsem_here)

                    def compute_chunk(n_valid):
                        @pl.loop(0, 8)
                        def _(j):
                            idxs = compute_group(crb, j, n_valid)
                            for ibf, h, iv in zip(ibs, hists, idxs):
                                ibf[pl.ds(j * 16, 16)] = iv
                                row = lax.shift_right_logical(iv, 7)
                                col = lax.bitwise_and(iv, 127)
                                plsc.addupdate_scatter(h, [row, col], one16)

                    @pl.when(k < NSUB - 1)
                    def _():
                        compute_chunk(128)

                    @pl.when(k == NSUB - 1)
                    def _():
                        compute_chunk(TAIL)

                    descs = [pltpu.async_copy(pfb, a.at[ib_], ssem, add=True)
                             for a, ib_ in zip(accs, ibs)]
                    for dd in descs:
                        dd.wait()

            issue_load(sid, pfbuf, crbuf, lsemA)

            @pl.loop(0, (QMAX + 1) // 2)
            def _(q2):
                k = sid + q2 * 32
                sub_iter(k, pfbuf, crbuf, ibufs, lsemA,
                         pfbuf2, crbuf2, lsemB)
                sub_iter(k + 16, pfbuf2, crbuf2, ibufs2, lsemB,
                         pfbuf, crbuf, lsemA)

            plsc.subcore_barrier()

            for h, cn in zip(hists, cnts):
                pltpu.sync_copy(h.at[pl.ds(0, NA)], cn.at[ia], add=True)
                if NBROWS > 0:
                    pltpu.sync_copy(h.at[pl.ds(NA, NBROWS)], cn.at[ib],
                                    add=True)

            plsc.subcore_barrier()

            @pl.loop(0, QROWS)
            def _(qr):
                r = sid + qr * 16

                @pl.when(r < NROWS)
                def _():
                    for a, cn, ou in zip(accs, cnts, outs):
                        d1 = pltpu.async_copy(
                            a.at[pl.ds(r * 128, 128)], abuf, lsemA)
                        d2 = pltpu.async_copy(
                            cn.at[pl.ds(r, 1)], cntrow, lsemB)
                        d1.wait()
                        d2.wait()

                        @pl.loop(0, 8)
                        def _(g):
                            cv = cntrow[0, pl.ds(g * 16, 16)]
                            rbuf[pl.ds(g * 16, 16)] = one16 / jnp.maximum(
                                cv, one16)

                        @pl.loop(0, 128)
                        def _(bn):
                            row = abuf[bn, pl.ds(0, C)]
                            e, o = plsc.unpack(
                                row, format=plsc.PackFormat.INTERLEAVED)
                            rec = plsc.load_gather(
                                rbuf, [jnp.broadcast_to(bn, (16,))])
                            col = jnp.broadcast_to(bn, (16,))
                            plsc.store_scatter(obuf, [evens, col], e * rec)
                            plsc.store_scatter(obuf, [odds, col], o * rec)

                        pltpu.sync_copy(obuf,
                                        ou.at[b, :, pl.ds(r * 128, 128)])

            plsc.subcore_barrier()

    return scale_kernel


_SCALES = ((100000, 128), (50000, 96), (25000, 64))


@functools.lru_cache(maxsize=None)
def _scale_kernels():
    return tuple(_make_scale_kernel(N, R) for (N, R) in _SCALES)


def kernel(p_f0, coord0, p_f1, coord1, p_f2, coord2):
    pfs = (p_f0, p_f1, p_f2)
    cds = (coord0, coord1, coord2)
    kernels = _scale_kernels()
    outs = []
    for i, (N, R) in enumerate(_SCALES):
        cr = jnp.transpose(cds[i] / jnp.float32(DENOM), (0, 2, 1))
        oxz, oxy, oyz = kernels[i](pfs[i].astype(jnp.bfloat16), cr)
        for o in (oxz, oxy, oyz):
            outs.append(o.reshape(B, C, R, R))
    return tuple(outs)
